# baseline (speedup 1.0000x reference)
import jax
import jax.numpy as jnp
from jax import lax
from jax.experimental import pallas as pl
from jax.experimental.pallas import tpu as pltpu
from jax.experimental.pallas import tpu_sc as plsc

N = 4096
K = 32
CUTOFF2 = 25.0
NC = 2
NS = 16
NW = NC * NS
CPW = N // NW
OPW = CPW * K
NG = N // 16
BIG = 1e18
HUGE = 3.0e30


def _splat(i):
  return jnp.full((16,), i, jnp.int32)


def _scalar(ref, i):
  return jax.lax.reduce_max(plsc.load_gather(ref, [_splat(i)]), axes=(0,))


def _sqrt16(x):
  i = plsc.bitcast(x, jnp.int32)
  i = jnp.int32(0x5F3759DF) - (i >> 1)
  y = plsc.bitcast(i, jnp.float32)
  for _ in range(4):
    y = y * (jnp.float32(1.5) - jnp.float32(0.5) * x * y * y)
  return x * y


def _bf16r(v):
  i = plsc.bitcast(v, jnp.int32)
  i = (i + jnp.int32(0x7FFF) + ((i >> 16) & jnp.int32(1))) & jnp.int32(-65536)
  return plsc.bitcast(i, jnp.float32)


def _merge16(ak, av, bk, bv):
  rk = lax.rev(bk, (0,))
  rv = lax.rev(bv, (0,))
  le = ak <= rk
  lok = jnp.where(le, ak, rk)
  lov = jnp.where(le, av, rv)
  hik = jnp.where(le, rk, ak)
  hiv = jnp.where(le, rv, av)
  lok, lov = plsc.sort_key_val(lok, lov)
  hik, hiv = plsc.sort_key_val(hik, hiv)
  return lok, lov, hik, hiv


def _body(x_hbm, y_hbm, z_hbm, b_hbm, glo_hbm, ghi_hbm,
          src_hbm, dst_hbm, w_hbm, vec_hbm,
          xv, yv, zv, bv, glc, ghc,
          srcv, dstv, wv, vecv):
  wid = lax.axis_index("s") * NC + lax.axis_index("c")
  pltpu.sync_copy(x_hbm, xv)
  pltpu.sync_copy(y_hbm, yv)
  pltpu.sync_copy(z_hbm, zv)
  pltpu.sync_copy(b_hbm, bv)
  pltpu.sync_copy(glo_hbm, glc)
  pltpu.sync_copy(ghi_hbm, ghc)

  iota16 = lax.iota(jnp.int32, 16)

  def center_body(jl, _):
    j = wid * CPW + jl
    jsplat = _splat(j)
    xj = plsc.load_gather(xv, [jsplat])
    yj = plsc.load_gather(yv, [jsplat])
    zj = plsc.load_gather(zv, [jsplat])
    bj = plsc.load_gather(bv, [jsplat])
    sqj = (xj * xj + yj * yj) + zj * zj
    xjb = _bf16r(xj)
    yjb = _bf16r(yj)
    zjb = _bf16r(zj)
    glo = _scalar(glc, j)
    ghi = _scalar(ghc, j)

    init = (jnp.full((16,), jnp.float32(BIG)), jsplat,
            jnp.full((16,), jnp.float32(BIG)), jsplat)

    def group_body(g, carry):
      t0k, t0v, t1k, t1v = carry
      bs = g * 16
      xc = xv[pl.ds(bs, 16)]
      yc = yv[pl.ds(bs, 16)]
      zc = zv[pl.ds(bs, 16)]
      bc = bv[pl.ds(bs, 16)]
      sqc = (xc * xc + yc * yc) + zc * zc
      dot = (_bf16r(xc) * xjb + _bf16r(yc) * yjb) + _bf16r(zc) * zjb
      d2 = jnp.maximum((sqc + sqj) - jnp.float32(2.0) * dot, jnp.float32(0.0))
      valid = (bc == bj) & (d2 <= jnp.float32(CUTOFF2))
      ck = jnp.where(valid, d2, jnp.float32(HUGE))
      cv = bs + iota16
      ck, cv = plsc.sort_key_val(ck, cv)
      t0k, t0v, hk, hv = _merge16(t0k, t0v, ck, cv)
      t1k, t1v, _, _ = _merge16(t1k, t1v, hk, hv)
      return (t0k, t0v, t1k, t1v)

    t0k, t0v, t1k, t1v = lax.fori_loop(glo, ghi, group_body, init)

    for half, sv in enumerate((t0v, t1v)):
      sx = plsc.load_gather(xv, [sv])
      sy = plsc.load_gather(yv, [sv])
      sz = plsc.load_gather(zv, [sv])
      dx = sx - xj
      dy = sy - yj
      dz = sz - zj
      s2 = (dx * dx + dy * dy) + dz * dz
      m = sv != jsplat
      wgt = jnp.where(m, _sqrt16(jnp.where(m, s2, jnp.float32(1.0))),
                      jnp.float32(0.0))
      slot = jl * K + half * 16
      srcv[pl.ds(slot, 16)] = sv
      dstv[pl.ds(slot, 16)] = jsplat
      wv[pl.ds(slot, 16)] = wgt
      rows = (slot + iota16) * 3
      plsc.store_scatter(vecv, [rows], dx)
      plsc.store_scatter(vecv, [rows + 1], dy)
      plsc.store_scatter(vecv, [rows + 2], dz)
    return 0

  lax.fori_loop(0, CPW, center_body, 0)

  obase = wid * OPW
  pltpu.sync_copy(srcv, src_hbm.at[pl.ds(obase, OPW)])
  pltpu.sync_copy(dstv, dst_hbm.at[pl.ds(obase, OPW)])
  pltpu.sync_copy(wv, w_hbm.at[pl.ds(obase, OPW)])
  pltpu.sync_copy(vecv, vec_hbm.at[pl.ds(obase * 3, OPW * 3)])


_sc_call = pl.kernel(
    _body,
    out_type=(
        jax.ShapeDtypeStruct((N * K,), jnp.int32),
        jax.ShapeDtypeStruct((N * K,), jnp.int32),
        jax.ShapeDtypeStruct((N * K,), jnp.float32),
        jax.ShapeDtypeStruct((N * K * 3,), jnp.float32),
    ),
    mesh=plsc.VectorSubcoreMesh(core_axis_name="c", subcore_axis_name="s"),
    compiler_params=pltpu.CompilerParams(needs_layout_passes=False),
    scratch_types=[
        pltpu.VMEM((N,), jnp.float32),
        pltpu.VMEM((N,), jnp.float32),
        pltpu.VMEM((N,), jnp.float32),
        pltpu.VMEM((N,), jnp.int32),
        pltpu.VMEM((N,), jnp.int32),
        pltpu.VMEM((N,), jnp.int32),
        pltpu.VMEM((OPW,), jnp.int32),
        pltpu.VMEM((OPW,), jnp.int32),
        pltpu.VMEM((OPW,), jnp.float32),
        pltpu.VMEM((OPW * 3,), jnp.float32),
    ],
)


@jax.jit
def kernel(pos, batch):
  x = pos[:, 0]
  y = pos[:, 1]
  z = pos[:, 2]
  b = batch.astype(jnp.int32)
  lo = jnp.searchsorted(b, b, side="left").astype(jnp.int32)
  hi = jnp.searchsorted(b, b, side="right").astype(jnp.int32)
  glo = lo // 16
  ghi = (hi + 15) // 16
  src, dst, w, vec = _sc_call(x, y, z, b, glo, ghi)
  return jnp.stack([src, dst]), w, vec.reshape(N * K, 3)


# speedup vs baseline: 2.1942x; 2.1942x over previous
import jax
import jax.numpy as jnp
from jax import lax
from jax.experimental import pallas as pl
from jax.experimental.pallas import tpu as pltpu
from jax.experimental.pallas import tpu_sc as plsc

N = 4096
K = 32
CUTOFF2 = 25.0
NC = 2
NS = 16
NW = NC * NS
CPW = N // NW
OPW = CPW * K
NG = N // 16
N_MOL = 32
BIG = 1e18
HUGE = 3.0e30


def _splat(i):
  return jnp.full((16,), i, jnp.int32)


def _scalar(ref, i):
  return jax.lax.reduce_max(plsc.load_gather(ref, [_splat(i)]), axes=(0,))


def _sqrt16(x):
  i = plsc.bitcast(x, jnp.int32)
  i = jnp.int32(0x5F3759DF) - (i >> 1)
  y = plsc.bitcast(i, jnp.float32)
  for _ in range(4):
    y = y * (jnp.float32(1.5) - jnp.float32(0.5) * x * y * y)
  return x * y


def _bf16r(v):
  i = plsc.bitcast(v, jnp.int32)
  i = (i + jnp.int32(0x7FFF) + ((i >> 16) & jnp.int32(1))) & jnp.int32(-65536)
  return plsc.bitcast(i, jnp.float32)


def _merge16(ak, av, bk, bv):
  rk = lax.rev(bk, (0,))
  rv = lax.rev(bv, (0,))
  le = ak <= rk
  lok = jnp.where(le, ak, rk)
  lov = jnp.where(le, av, rv)
  hik = jnp.where(le, rk, ak)
  hiv = jnp.where(le, rv, av)
  lok, lov = plsc.sort_key_val(lok, lov)
  hik, hiv = plsc.sort_key_val(hik, hiv)
  return lok, lov, hik, hiv


def _body(x_hbm, y_hbm, z_hbm, b_hbm, glo_hbm, ghi_hbm,
          src_hbm, dst_hbm, w_hbm, vec_hbm,
          xv, yv, zv, bv, glc, ghc,
          srcv, dstv, wv, vecv):
  wid = lax.axis_index("s") * NC + lax.axis_index("c")
  pltpu.sync_copy(x_hbm, xv)
  pltpu.sync_copy(y_hbm, yv)
  pltpu.sync_copy(z_hbm, zv)
  pltpu.sync_copy(b_hbm, bv)
  pltpu.sync_copy(glo_hbm, glc)
  pltpu.sync_copy(ghi_hbm, ghc)

  iota16 = lax.iota(jnp.int32, 16)

  def center_body(jl, _):
    j = wid * CPW + jl
    jsplat = _splat(j)
    xj = plsc.load_gather(xv, [jsplat])
    yj = plsc.load_gather(yv, [jsplat])
    zj = plsc.load_gather(zv, [jsplat])
    bj = plsc.load_gather(bv, [jsplat])
    sqj = (xj * xj + yj * yj) + zj * zj
    xjb = _bf16r(xj)
    yjb = _bf16r(yj)
    zjb = _bf16r(zj)
    glo = _scalar(glc, j)
    ghi = _scalar(ghc, j)

    init = (jnp.full((16,), jnp.float32(BIG)), jsplat,
            jnp.full((16,), jnp.float32(BIG)), jsplat)

    def group_body(g, carry):
      t0k, t0v, t1k, t1v = carry
      bs = g * 16
      xc = xv[pl.ds(bs, 16)]
      yc = yv[pl.ds(bs, 16)]
      zc = zv[pl.ds(bs, 16)]
      bc = bv[pl.ds(bs, 16)]
      sqc = (xc * xc + yc * yc) + zc * zc
      dot = (_bf16r(xc) * xjb + _bf16r(yc) * yjb) + _bf16r(zc) * zjb
      d2 = jnp.maximum((sqc + sqj) - jnp.float32(2.0) * dot, jnp.float32(0.0))
      valid = (bc == bj) & (d2 <= jnp.float32(CUTOFF2))
      ck = jnp.where(valid, d2, jnp.float32(HUGE))
      cv = bs + iota16
      ck, cv = plsc.sort_key_val(ck, cv)
      t0k, t0v, hk, hv = _merge16(t0k, t0v, ck, cv)
      t1k, t1v, _, _ = _merge16(t1k, t1v, hk, hv)
      return (t0k, t0v, t1k, t1v)

    t0k, t0v, t1k, t1v = lax.fori_loop(glo, ghi, group_body, init)

    for half, sv in enumerate((t0v, t1v)):
      sx = plsc.load_gather(xv, [sv])
      sy = plsc.load_gather(yv, [sv])
      sz = plsc.load_gather(zv, [sv])
      dx = sx - xj
      dy = sy - yj
      dz = sz - zj
      s2 = (dx * dx + dy * dy) + dz * dz
      m = sv != jsplat
      wgt = jnp.where(m, _sqrt16(jnp.where(m, s2, jnp.float32(1.0))),
                      jnp.float32(0.0))
      slot = jl * K + half * 16
      srcv[pl.ds(slot, 16)] = sv
      dstv[pl.ds(slot, 16)] = jsplat
      wv[pl.ds(slot, 16)] = wgt
      rows = (slot + iota16) * 3
      plsc.store_scatter(vecv, [rows], dx)
      plsc.store_scatter(vecv, [rows + 1], dy)
      plsc.store_scatter(vecv, [rows + 2], dz)
    return 0

  lax.fori_loop(0, CPW, center_body, 0)

  obase = wid * OPW
  pltpu.sync_copy(srcv, src_hbm.at[pl.ds(obase, OPW)])
  pltpu.sync_copy(dstv, dst_hbm.at[pl.ds(obase, OPW)])
  pltpu.sync_copy(wv, w_hbm.at[pl.ds(obase, OPW)])
  pltpu.sync_copy(vecv, vec_hbm.at[pl.ds(obase * 3, OPW * 3)])


_sc_call = pl.kernel(
    _body,
    out_type=(
        jax.ShapeDtypeStruct((N * K,), jnp.int32),
        jax.ShapeDtypeStruct((N * K,), jnp.int32),
        jax.ShapeDtypeStruct((N * K,), jnp.float32),
        jax.ShapeDtypeStruct((N * K * 3,), jnp.float32),
    ),
    mesh=plsc.VectorSubcoreMesh(core_axis_name="c", subcore_axis_name="s"),
    compiler_params=pltpu.CompilerParams(needs_layout_passes=False),
    scratch_types=[
        pltpu.VMEM((N,), jnp.float32),
        pltpu.VMEM((N,), jnp.float32),
        pltpu.VMEM((N,), jnp.float32),
        pltpu.VMEM((N,), jnp.int32),
        pltpu.VMEM((N,), jnp.int32),
        pltpu.VMEM((N,), jnp.int32),
        pltpu.VMEM((OPW,), jnp.int32),
        pltpu.VMEM((OPW,), jnp.int32),
        pltpu.VMEM((OPW,), jnp.float32),
        pltpu.VMEM((OPW * 3,), jnp.float32),
    ],
)


@jax.jit
def kernel(pos, batch):
  x = pos[:, 0]
  y = pos[:, 1]
  z = pos[:, 2]
  b = batch.astype(jnp.int32)
  counts = jnp.sum(b[:, None] == jnp.arange(N_MOL, dtype=jnp.int32)[None, :],
                   axis=0, dtype=jnp.int32)
  starts = jnp.concatenate([jnp.zeros((1,), jnp.int32), jnp.cumsum(counts)])
  lo = starts[b]
  hi = starts[b + 1]
  glo = lo // 16
  ghi = (hi + 15) // 16
  src, dst, w, vec = _sc_call(x, y, z, b, glo, ghi)
  return jnp.stack([src, dst]), w, vec.reshape(N * K, 3)


# speedup vs baseline: 2.8012x; 1.2767x over previous
import jax
import jax.numpy as jnp
from jax import lax
from jax.experimental import pallas as pl
from jax.experimental.pallas import tpu as pltpu
from jax.experimental.pallas import tpu_sc as plsc

N = 4096
K = 32
CUTOFF2 = 25.0
N_MOL = 32
NC = 2
NS = 16
NW = NC * NS
CPW = N // NW
OPW = CPW * K
BIG = 1e18
HUGE = 3.0e30


def _splat(i):
  return jnp.full((16,), i, jnp.int32)


def _scalar16(v):
  return jax.lax.reduce_max(v, axes=(0,))


def _sqrt16(x):
  i = plsc.bitcast(x, jnp.int32)
  i = jnp.int32(0x5F3759DF) - (i >> 1)
  y = plsc.bitcast(i, jnp.float32)
  for _ in range(4):
    y = y * (jnp.float32(1.5) - jnp.float32(0.5) * x * y * y)
  return x * y


def _bf16r(v):
  i = plsc.bitcast(v, jnp.int32)
  i = (i + jnp.int32(0x7FFF) + ((i >> 16) & jnp.int32(1))) & jnp.int32(-65536)
  return plsc.bitcast(i, jnp.float32)


def _merge16(ak, av, bk, bv):
  rk = lax.rev(bk, (0,))
  rv = lax.rev(bv, (0,))
  le = ak <= rk
  lok = jnp.where(le, ak, rk)
  lov = jnp.where(le, av, rv)
  hik = jnp.where(le, rk, ak)
  hiv = jnp.where(le, rv, av)
  lok, lov = plsc.sort_key_val(lok, lov)
  hik, hiv = plsc.sort_key_val(hik, hiv)
  return lok, lov, hik, hiv


def _body(x_hbm, y_hbm, z_hbm, b_hbm, st_hbm,
          src_hbm, dst_hbm, w_hbm, vec_hbm,
          xv, yv, zv, bv, stv, srcv, dstv, wv, vecv):
  wid = lax.axis_index("s") * NC + lax.axis_index("c")
  pltpu.sync_copy(x_hbm, xv)
  pltpu.sync_copy(y_hbm, yv)
  pltpu.sync_copy(z_hbm, zv)
  pltpu.sync_copy(b_hbm, bv)
  pltpu.sync_copy(st_hbm, stv)

  iota16 = lax.iota(jnp.int32, 16)

  def center_body(jl, _):
    j = wid * CPW + jl
    jsplat = _splat(j)
    xj = plsc.load_gather(xv, [jsplat])
    yj = plsc.load_gather(yv, [jsplat])
    zj = plsc.load_gather(zv, [jsplat])
    bj = plsc.load_gather(bv, [jsplat])
    sqj = (xj * xj + yj * yj) + zj * zj
    xjb = _bf16r(xj)
    yjb = _bf16r(yj)
    zjb = _bf16r(zj)
    lo = _scalar16(plsc.load_gather(stv, [bj]))
    hi = _scalar16(plsc.load_gather(stv, [bj + 1]))
    glo = lo >> 4
    ghi = (hi + 15) >> 4

    init = (jnp.full((16,), jnp.float32(BIG)), jsplat,
            jnp.full((16,), jnp.float32(BIG)), jsplat)

    def group_body(g, carry):
      t0k, t0v, t1k, t1v = carry
      bs = g * 16
      xc = xv[pl.ds(bs, 16)]
      yc = yv[pl.ds(bs, 16)]
      zc = zv[pl.ds(bs, 16)]
      bc = bv[pl.ds(bs, 16)]
      sqc = (xc * xc + yc * yc) + zc * zc
      dot = (_bf16r(xc) * xjb + _bf16r(yc) * yjb) + _bf16r(zc) * zjb
      d2 = jnp.maximum((sqc + sqj) - jnp.float32(2.0) * dot, jnp.float32(0.0))
      valid = (bc == bj) & (d2 <= jnp.float32(CUTOFF2))
      ck = jnp.where(valid, d2, jnp.float32(HUGE))
      cv = bs + iota16
      ck, cv = plsc.sort_key_val(ck, cv)
      t0k, t0v, hk, hv = _merge16(t0k, t0v, ck, cv)
      t1k, t1v, _, _ = _merge16(t1k, t1v, hk, hv)
      return (t0k, t0v, t1k, t1v)

    t0k, t0v, t1k, t1v = lax.fori_loop(glo, ghi, group_body, init)

    for half, sv in enumerate((t0v, t1v)):
      sx = plsc.load_gather(xv, [sv])
      sy = plsc.load_gather(yv, [sv])
      sz = plsc.load_gather(zv, [sv])
      dx = sx - xj
      dy = sy - yj
      dz = sz - zj
      s2 = (dx * dx + dy * dy) + dz * dz
      m = sv != jsplat
      wgt = jnp.where(m, _sqrt16(jnp.where(m, s2, jnp.float32(1.0))),
                      jnp.float32(0.0))
      slot = jl * K + half * 16
      srcv[pl.ds(slot, 16)] = sv
      dstv[pl.ds(slot, 16)] = jsplat
      wv[pl.ds(slot, 16)] = wgt
      rows = (slot + iota16) * 3
      plsc.store_scatter(vecv, [rows], dx)
      plsc.store_scatter(vecv, [rows + 1], dy)
      plsc.store_scatter(vecv, [rows + 2], dz)
    return 0

  lax.fori_loop(0, CPW, center_body, 0)

  obase = wid * OPW
  pltpu.sync_copy(srcv, src_hbm.at[pl.ds(obase, OPW)])
  pltpu.sync_copy(dstv, dst_hbm.at[pl.ds(obase, OPW)])
  pltpu.sync_copy(wv, w_hbm.at[pl.ds(obase, OPW)])
  pltpu.sync_copy(vecv, vec_hbm.at[pl.ds(obase * 3, OPW * 3)])


_sc_call = pl.kernel(
    _body,
    out_type=(
        jax.ShapeDtypeStruct((N * K,), jnp.int32),
        jax.ShapeDtypeStruct((N * K,), jnp.int32),
        jax.ShapeDtypeStruct((N * K,), jnp.float32),
        jax.ShapeDtypeStruct((N * K * 3,), jnp.float32),
    ),
    mesh=plsc.VectorSubcoreMesh(core_axis_name="c", subcore_axis_name="s"),
    compiler_params=pltpu.CompilerParams(needs_layout_passes=False),
    scratch_types=[
        pltpu.VMEM((N,), jnp.float32),
        pltpu.VMEM((N,), jnp.float32),
        pltpu.VMEM((N,), jnp.float32),
        pltpu.VMEM((N,), jnp.int32),
        pltpu.VMEM((128,), jnp.int32),
        pltpu.VMEM((OPW,), jnp.int32),
        pltpu.VMEM((OPW,), jnp.int32),
        pltpu.VMEM((OPW,), jnp.float32),
        pltpu.VMEM((OPW * 3,), jnp.float32),
    ],
)


@jax.jit
def kernel(pos, batch):
  x = pos[:, 0]
  y = pos[:, 1]
  z = pos[:, 2]
  b = batch.astype(jnp.int32)
  counts = jnp.sum(b[:, None] == jnp.arange(N_MOL, dtype=jnp.int32)[None, :],
                   axis=0, dtype=jnp.int32)
  starts = jnp.zeros((128,), jnp.int32).at[1:N_MOL + 1].set(jnp.cumsum(counts))
  src, dst, w, vec = _sc_call(x, y, z, b, starts)
  return jnp.stack([src, dst]), w, vec.reshape(N * K, 3)
